# baseline (speedup 1.0000x reference)
import functools

import jax
import jax.numpy as jnp
from jax import lax
from jax.experimental import pallas as pl
from jax.experimental.pallas import tpu as pltpu
from jax.experimental.pallas import tpu_sc as plsc

N = 10000
E = 320000
D_IN = 128
D_HID = 16
D_OUT = 64

NC = 2
NS = 16
NW = NC * NS
EPT = E // NW
B = 80
NB = EPT // B
N_PAD = 10240
RPT = N_PAD // NS

_MESH = dict(core_axis_name="c", subcore_axis_name="s", num_cores=NC,
             num_subcores=NS)

_SC_CACHE = {}

_SC_PARAMS = pltpu.CompilerParams(use_tc_tiling_on_sc=False)


def _make_edge_pass(d):
  @functools.partial(
      pl.kernel,
      out_type=jax.ShapeDtypeStruct((NC, N_PAD, d), jnp.float32),
      mesh=plsc.VectorSubcoreMesh(**_MESH),
      scratch_types=[
          pltpu.VMEM((NB, B), jnp.int32),
          pltpu.VMEM((NB, B), jnp.int32),
          pltpu.VMEM((B, d), jnp.float32),
          pltpu.VMEM((B, d), jnp.float32),
          pltpu.SemaphoreType.DMA,
          pltpu.SemaphoreType.DMA,
          pltpu.VMEM_SHARED((N_PAD, d), jnp.float32),
      ],
      compiler_params=_SC_PARAMS,
  )
  def edge_pass(y_hbm, src_hbm, dst_hbm, out_hbm, srcv, dstv, buf0,
                buf1, sem0, sem1, acc):
    cid = lax.axis_index("c")
    sid = lax.axis_index("s")
    wid = sid * NC + cid
    pltpu.sync_copy(src_hbm.at[wid], srcv)
    pltpu.sync_copy(dst_hbm.at[wid], dstv)

    def zrow(i, c):
      for cc in range(d // 16):
        buf0[i, pl.ds(cc * 16, 16)] = jnp.zeros((16,), jnp.float32)
      return c

    lax.fori_loop(0, B, zrow, 0, unroll=False)
    for r in range(RPT // B):
      pltpu.sync_copy(buf0, acc.at[pl.ds(sid * RPT + r * B, B)])
    plsc.subcore_barrier()

    bufs = (buf0, buf1)
    sems = (sem0, sem1)

    def gather(j, b):
      pltpu.async_copy(y_hbm.at[srcv.at[j]], bufs[b], sems[b])

    def gwait(j, b):
      pltpu.make_async_copy(y_hbm.at[srcv.at[j]], bufs[b], sems[b]).wait()

    def scat(j, b):
      pltpu.sync_copy(bufs[b], acc.at[dstv.at[j]], add=True)

    gather(0, 0)

    def step(i, carry):
      j0 = 2 * i
      gather(j0 + 1, 1)
      gwait(j0, 0)
      scat(j0, 0)
      gather(j0 + 2, 0)
      gwait(j0 + 1, 1)
      scat(j0 + 1, 1)
      return carry

    lax.fori_loop(0, (NB - 1) // 2, step, 0, unroll=False)
    gwait(NB - 1, 0)
    scat(NB - 1, 0)
    plsc.subcore_barrier()
    pltpu.sync_copy(acc.at[pl.ds(sid * RPT, RPT)],
                    out_hbm.at[cid, pl.ds(sid * RPT, RPT)])

  return edge_pass


def _make_deg_pass():
  @functools.partial(
      pl.kernel,
      out_type=jax.ShapeDtypeStruct((NC, N_PAD, D_HID), jnp.float32),
      mesh=plsc.VectorSubcoreMesh(**_MESH),
      scratch_types=[
          pltpu.VMEM((NB, B), jnp.int32),
          pltpu.VMEM((B, D_HID), jnp.float32),
          pltpu.SemaphoreType.DMA,
          pltpu.VMEM_SHARED((N_PAD, D_HID), jnp.float32),
      ],
      compiler_params=_SC_PARAMS,
  )
  def deg_pass(dst_hbm, out_hbm, dstv, buf, sem, acc):
    cid = lax.axis_index("c")
    sid = lax.axis_index("s")
    wid = sid * NC + cid
    pltpu.sync_copy(dst_hbm.at[wid], dstv)

    def fill(val):
      def frow(i, c):
        buf[i, :] = jnp.full((16,), val, jnp.float32)
        return c
      lax.fori_loop(0, B, frow, 0, unroll=False)

    fill(0.0)
    for r in range(RPT // B):
      pltpu.sync_copy(buf, acc.at[pl.ds(sid * RPT + r * B, B)])
    fill(1.0)
    plsc.subcore_barrier()

    def fire(j, c):
      pltpu.async_copy(buf, acc.at[dstv.at[j]], sem, add=True)
      return c

    def drain(j, c):
      pltpu.make_async_copy(buf, acc.at[dstv.at[j]], sem).wait()
      return c

    lax.fori_loop(0, NB, fire, 0, unroll=False)
    lax.fori_loop(0, NB, drain, 0, unroll=False)
    plsc.subcore_barrier()
    pltpu.sync_copy(acc.at[pl.ds(sid * RPT, RPT)],
                    out_hbm.at[cid, pl.ds(sid * RPT, RPT)])

  return deg_pass


def _sc_kernels():
  if not _SC_CACHE:
    _SC_CACHE["edge16"] = _make_edge_pass(D_HID)
    _SC_CACHE["deg"] = _make_deg_pass()
  return _SC_CACHE["deg"], _SC_CACHE["edge16"]


_NF16 = N * D_HID // 128


def _tca_body(x8, w1b, dg, yo):
  deg = dg[0, :_NF16] + dg[1, :_NF16] + 1.0
  h = jnp.dot(x8[...], w1b[...], preferred_element_type=jnp.float32)
  yo[...] = h * lax.rsqrt(deg)


def _tcb_body(dg, p1f, y1f, b1f, ho):
  dinv = lax.rsqrt(dg[0, :_NF16] + dg[1, :_NF16] + 1.0)
  h = jnp.maximum(
      (p1f[0, :_NF16] + p1f[1, :_NF16] + y1f[...]) * dinv + b1f[...], 0.0)
  ho[...] = h * dinv


def _tcc_body(dg, p2f, hdf, w2b, b2, o):
  dinv = lax.rsqrt(dg[0, :_NF16] + dg[1, :_NF16] + 1.0)
  f = (p2f[0, :_NF16] + p2f[1, :_NF16] + hdf[...]) * dinv
  aa = jnp.dot(f, w2b[...], preferred_element_type=jnp.float32)
  for c in range(8):
    a = aa[:, c * 64:(c + 1) * 64] + b2[...]
    m = jnp.max(a, axis=1, keepdims=True)
    ex = jnp.exp(a - m)
    o[:, c, :] = a - (jnp.log(jnp.sum(ex, axis=1, keepdims=True)) + m)


_tca = pl.pallas_call(
    _tca_body,
    out_shape=jax.ShapeDtypeStruct((_NF16, 128), jnp.float32),
)

_tcb = pl.pallas_call(
    _tcb_body,
    out_shape=jax.ShapeDtypeStruct((_NF16, 128), jnp.float32),
)

_tcc = pl.pallas_call(
    _tcc_body,
    out_shape=jax.ShapeDtypeStruct((_NF16, 8, D_OUT), jnp.float32),
)


def kernel(x, edge_index, W1, b1, W2, b2):
  ei = edge_index.astype(jnp.int32)
  src3 = ei[0].reshape(NW, NB, B)
  dst3 = ei[1].reshape(NW, NB, B)

  _deg, _edge16 = _sc_kernels()
  degp = _deg(dst3)
  degf = degp.reshape(NC, 1280, 128)

  x8 = x.reshape(1250, 1024)
  w1b = jnp.kron(jnp.eye(8, dtype=jnp.float32), W1)
  y1f = _tca(x8, w1b, degf)

  p1 = _edge16(y1f.reshape(N, D_HID), src3, dst3)
  p1f = p1.reshape(NC, 1280, 128)
  b1f = jnp.tile(b1, 8).reshape(1, 128)
  hdf = _tcb(degf, p1f, y1f, b1f)

  p2 = _edge16(hdf.reshape(N, D_HID), src3, dst3)
  p2f = p2.reshape(NC, 1280, 128)
  w2b = jnp.kron(jnp.eye(8, dtype=jnp.float32), W2)
  out3 = _tcc(degf, p2f, hdf, w2b, b2.reshape(1, D_OUT))
  return out3.reshape(N, D_OUT)


# speedup vs baseline: 1.0219x; 1.0219x over previous
import functools

import jax
import jax.numpy as jnp
from jax import lax
from jax.experimental import pallas as pl
from jax.experimental.pallas import tpu as pltpu
from jax.experimental.pallas import tpu_sc as plsc

N = 10000
E = 320000
D_IN = 128
D_HID = 16
D_OUT = 64

NC = 2
NS = 16
NW = NC * NS
EPT = E // NW
B = 80
NB = EPT // B
N_PAD = 10240
RPT = N_PAD // NS

_MESH = dict(core_axis_name="c", subcore_axis_name="s", num_cores=NC,
             num_subcores=NS)

_SC_CACHE = {}

_SC_PARAMS = pltpu.CompilerParams(use_tc_tiling_on_sc=False)


def _make_edge_pass(d):
  @functools.partial(
      pl.kernel,
      out_type=jax.ShapeDtypeStruct((NC, N_PAD, d), jnp.float32),
      mesh=plsc.VectorSubcoreMesh(**_MESH),
      scratch_types=[
          pltpu.VMEM((NB, B), jnp.int32),
          pltpu.VMEM((NB, B), jnp.int32),
          pltpu.VMEM((B, d), jnp.float32),
          pltpu.VMEM((B, d), jnp.float32),
          pltpu.SemaphoreType.DMA,
          pltpu.SemaphoreType.DMA,
          pltpu.VMEM_SHARED((N_PAD, d), jnp.float32),
      ],
      compiler_params=_SC_PARAMS,
  )
  def edge_pass(y_hbm, src_hbm, dst_hbm, out_hbm, srcv, dstv, buf0,
                buf1, sem0, sem1, acc):
    cid = lax.axis_index("c")
    sid = lax.axis_index("s")
    wid = sid * NC + cid
    pltpu.sync_copy(src_hbm.at[wid], srcv)
    pltpu.sync_copy(dst_hbm.at[wid], dstv)

    def zrow(i, c):
      for cc in range(d // 16):
        buf0[i, pl.ds(cc * 16, 16)] = jnp.zeros((16,), jnp.float32)
      return c

    lax.fori_loop(0, B, zrow, 0, unroll=False)
    for r in range(RPT // B):
      pltpu.sync_copy(buf0, acc.at[pl.ds(sid * RPT + r * B, B)])
    plsc.subcore_barrier()

    bufs = (buf0, buf1)
    sems = (sem0, sem1)

    def gather(j, b):
      pltpu.async_copy(y_hbm.at[srcv.at[j]], bufs[b], sems[b])

    def gwait(j, b):
      pltpu.make_async_copy(y_hbm.at[srcv.at[j]], bufs[b], sems[b]).wait()

    def scat(j, b):
      pltpu.sync_copy(bufs[b], acc.at[dstv.at[j]], add=True)

    gather(0, 0)

    def step(i, carry):
      j0 = 2 * i
      gather(j0 + 1, 1)
      gwait(j0, 0)
      scat(j0, 0)
      gather(j0 + 2, 0)
      gwait(j0 + 1, 1)
      scat(j0 + 1, 1)
      return carry

    lax.fori_loop(0, (NB - 1) // 2, step, 0, unroll=False)
    gwait(NB - 1, 0)
    scat(NB - 1, 0)
    plsc.subcore_barrier()
    pltpu.sync_copy(acc.at[pl.ds(sid * RPT, RPT)],
                    out_hbm.at[cid, pl.ds(sid * RPT, RPT)])

  return edge_pass


def _make_deg_pass():
  @functools.partial(
      pl.kernel,
      out_type=jax.ShapeDtypeStruct((NC, N_PAD, D_HID), jnp.float32),
      mesh=plsc.VectorSubcoreMesh(**_MESH),
      scratch_types=[
          pltpu.VMEM((NB, B), jnp.int32),
          pltpu.VMEM((B, D_HID), jnp.float32),
          pltpu.SemaphoreType.DMA,
          pltpu.VMEM_SHARED((N_PAD, D_HID), jnp.float32),
      ],
      compiler_params=_SC_PARAMS,
  )
  def deg_pass(dst_hbm, out_hbm, dstv, buf, sem, acc):
    cid = lax.axis_index("c")
    sid = lax.axis_index("s")
    wid = sid * NC + cid
    pltpu.sync_copy(dst_hbm.at[wid], dstv)

    def fill(val):
      def frow(i, c):
        buf[i, :] = jnp.full((16,), val, jnp.float32)
        return c
      lax.fori_loop(0, B, frow, 0, unroll=False)

    fill(0.0)
    for r in range(RPT // B):
      pltpu.sync_copy(buf, acc.at[pl.ds(sid * RPT + r * B, B)])
    fill(1.0)
    plsc.subcore_barrier()

    def fire(j, c):
      pltpu.async_copy(buf, acc.at[dstv.at[j]], sem, add=True)
      return c

    def drain(j, c):
      pltpu.make_async_copy(buf, acc.at[dstv.at[j]], sem).wait()
      return c

    lax.fori_loop(0, NB, fire, 0, unroll=False)
    lax.fori_loop(0, NB, drain, 0, unroll=False)
    plsc.subcore_barrier()
    pltpu.sync_copy(acc.at[pl.ds(sid * RPT, RPT)],
                    out_hbm.at[cid, pl.ds(sid * RPT, RPT)])

  return deg_pass


def _sc_kernels():
  if not _SC_CACHE:
    _SC_CACHE["edge16"] = _make_edge_pass(D_HID)
    _SC_CACHE["deg"] = _make_deg_pass()
  return _SC_CACHE["deg"], _SC_CACHE["edge16"]


_NF16 = N * D_HID // 128


def _tca_body(x8, w1b, dg, yo):
  deg = dg[0, :_NF16] + dg[1, :_NF16] + 1.0
  h = jnp.dot(x8[...], w1b[...], preferred_element_type=jnp.float32)
  yo[...] = h * lax.rsqrt(deg)


def _tcb_body(dg, p1f, y1f, b1f, ho):
  dinv = lax.rsqrt(dg[0, :_NF16] + dg[1, :_NF16] + 1.0)
  h = jnp.maximum(
      (p1f[0, :_NF16] + p1f[1, :_NF16] + y1f[...]) * dinv + b1f[...], 0.0)
  ho[...] = h * dinv


def _tcc_body(dg, p2f, hdf, w2, b2, o):
  dinv = lax.rsqrt(dg[0, :_NF16] + dg[1, :_NF16] + 1.0)
  f = (p2f[0, :_NF16] + p2f[1, :_NF16] + hdf[...]) * dinv
  for c in range(8):
    fc = f[:, c * 16:(c + 1) * 16]
    a = jnp.dot(fc, w2[...], preferred_element_type=jnp.float32) + b2[...]
    m = jnp.max(a, axis=1, keepdims=True)
    ex = jnp.exp(a - m)
    o[:, c, :] = a - (jnp.log(jnp.sum(ex, axis=1, keepdims=True)) + m)


_tca = pl.pallas_call(
    _tca_body,
    out_shape=jax.ShapeDtypeStruct((_NF16, 128), jnp.float32),
)

_tcb = pl.pallas_call(
    _tcb_body,
    out_shape=jax.ShapeDtypeStruct((_NF16, 128), jnp.float32),
)

_tcc = pl.pallas_call(
    _tcc_body,
    out_shape=jax.ShapeDtypeStruct((_NF16, 8, D_OUT), jnp.float32),
)


def kernel(x, edge_index, W1, b1, W2, b2):
  ei = edge_index.astype(jnp.int32)
  dst3 = ei[1].reshape(NW, NB, B)

  _deg, _edge16 = _sc_kernels()
  degp = _deg(dst3)
  ei0, _ = lax.optimization_barrier((ei[0], dst3))
  src3 = ei0.reshape(NW, NB, B)
  degf = degp.reshape(NC, 1280, 128)

  x8 = x.reshape(1250, 1024)
  w1b = jnp.kron(jnp.eye(8, dtype=jnp.float32), W1)
  y1f = _tca(x8, w1b, degf)

  p1 = _edge16(y1f.reshape(N, D_HID), src3, dst3)
  p1f = p1.reshape(NC, 1280, 128)
  b1f = jnp.tile(b1, 8).reshape(1, 128)
  hdf = _tcb(degf, p1f, y1f, b1f)

  p2 = _edge16(hdf.reshape(N, D_HID), src3, dst3)
  p2f = p2.reshape(NC, 1280, 128)
  out3 = _tcc(degf, p2f, hdf, W2, b2.reshape(1, D_OUT))
  return out3.reshape(N, D_OUT)


# speedup vs baseline: 1.0227x; 1.0008x over previous
import functools

import jax
import jax.numpy as jnp
from jax import lax
from jax.experimental import pallas as pl
from jax.experimental.pallas import tpu as pltpu
from jax.experimental.pallas import tpu_sc as plsc

N = 10000
E = 320000
D_IN = 128
D_HID = 16
D_OUT = 64

NC = 2
NS = 16
NW = NC * NS
EPT = E // NW
B = 80
NB = EPT // B
N_PAD = 10240
RPT = N_PAD // NS

_MESH = dict(core_axis_name="c", subcore_axis_name="s", num_cores=NC,
             num_subcores=NS)

_SC_CACHE = {}

_SC_PARAMS = pltpu.CompilerParams(use_tc_tiling_on_sc=False)


def _make_edge_pass(d):
  @functools.partial(
      pl.kernel,
      out_type=jax.ShapeDtypeStruct((NC, N_PAD, d), jnp.float32),
      mesh=plsc.VectorSubcoreMesh(**_MESH),
      scratch_types=[
          pltpu.VMEM((NB, B), jnp.int32),
          pltpu.VMEM((NB, B), jnp.int32),
          pltpu.VMEM((B, d), jnp.float32),
          pltpu.VMEM((B, d), jnp.float32),
          pltpu.SemaphoreType.DMA,
          pltpu.SemaphoreType.DMA,
          pltpu.VMEM_SHARED((N_PAD, d), jnp.float32),
      ],
      compiler_params=_SC_PARAMS,
  )
  def edge_pass(y_hbm, src_hbm, dst_hbm, out_hbm, srcv, dstv, buf0,
                buf1, sem0, sem1, acc):
    cid = lax.axis_index("c")
    sid = lax.axis_index("s")
    wid = sid * NC + cid
    pltpu.sync_copy(src_hbm.at[wid], srcv)
    pltpu.sync_copy(dst_hbm.at[wid], dstv)

    def zrow(i, c):
      for cc in range(d // 16):
        buf0[i, pl.ds(cc * 16, 16)] = jnp.zeros((16,), jnp.float32)
      return c

    lax.fori_loop(0, B, zrow, 0, unroll=False)
    for r in range(RPT // B):
      pltpu.sync_copy(buf0, acc.at[pl.ds(sid * RPT + r * B, B)])
    plsc.subcore_barrier()

    bufs = (buf0, buf1)
    sems = (sem0, sem1)

    def gather(j, b):
      pltpu.async_copy(y_hbm.at[srcv.at[j]], bufs[b], sems[b])

    def gwait(j, b):
      pltpu.make_async_copy(y_hbm.at[srcv.at[j]], bufs[b], sems[b]).wait()

    def scat(j, b):
      pltpu.sync_copy(bufs[b], acc.at[dstv.at[j]], add=True)

    gather(0, 0)

    def step(i, carry):
      j0 = 2 * i
      gather(j0 + 1, 1)
      gwait(j0, 0)
      scat(j0, 0)
      gather(j0 + 2, 0)
      gwait(j0 + 1, 1)
      scat(j0 + 1, 1)
      return carry

    lax.fori_loop(0, (NB - 1) // 2, step, 0, unroll=False)
    gwait(NB - 1, 0)
    scat(NB - 1, 0)
    plsc.subcore_barrier()
    pltpu.sync_copy(acc.at[pl.ds(sid * RPT, RPT)],
                    out_hbm.at[cid, pl.ds(sid * RPT, RPT)])

  return edge_pass


def _make_deg_pass():
  @functools.partial(
      pl.kernel,
      out_type=jax.ShapeDtypeStruct((NC, N_PAD, D_HID), jnp.float32),
      mesh=plsc.VectorSubcoreMesh(**_MESH),
      scratch_types=[
          pltpu.VMEM((NB, B), jnp.int32),
          pltpu.VMEM((B, D_HID), jnp.float32),
          pltpu.SemaphoreType.DMA,
          pltpu.VMEM_SHARED((N_PAD, D_HID), jnp.float32),
      ],
      compiler_params=_SC_PARAMS,
  )
  def deg_pass(dst_hbm, out_hbm, dstv, buf, sem, acc):
    cid = lax.axis_index("c")
    sid = lax.axis_index("s")
    wid = sid * NC + cid
    pltpu.sync_copy(dst_hbm.at[wid], dstv)

    def fill(val):
      def frow(i, c):
        buf[i, :] = jnp.full((16,), val, jnp.float32)
        return c
      lax.fori_loop(0, B, frow, 0, unroll=False)

    fill(0.0)
    for r in range(RPT // B):
      pltpu.sync_copy(buf, acc.at[pl.ds(sid * RPT + r * B, B)])
    fill(1.0)
    plsc.subcore_barrier()

    def fire(j, c):
      pltpu.async_copy(buf, acc.at[dstv.at[j]], sem, add=True)
      return c

    def drain(j, c):
      pltpu.make_async_copy(buf, acc.at[dstv.at[j]], sem).wait()
      return c

    lax.fori_loop(0, NB, fire, 0, unroll=False)
    lax.fori_loop(0, NB, drain, 0, unroll=False)
    plsc.subcore_barrier()
    pltpu.sync_copy(acc.at[pl.ds(sid * RPT, RPT)],
                    out_hbm.at[cid, pl.ds(sid * RPT, RPT)])

  return deg_pass


def _sc_kernels():
  if not _SC_CACHE:
    _SC_CACHE["edge16"] = _make_edge_pass(D_HID)
    _SC_CACHE["deg"] = _make_deg_pass()
  return _SC_CACHE["deg"], _SC_CACHE["edge16"]


_NF16 = N * D_HID // 128


def _tca_body(x8, w1b, dg, yo):
  deg = dg[0, :_NF16] + dg[1, :_NF16] + 1.0
  h = jnp.dot(x8[...], w1b[...], preferred_element_type=jnp.float32)
  yo[...] = h * lax.rsqrt(deg)


def _tcb_body(dg, p1f, y1f, b1f, ho):
  dinv = lax.rsqrt(dg[0, :_NF16] + dg[1, :_NF16] + 1.0)
  h = jnp.maximum(
      (p1f[0, :_NF16] + p1f[1, :_NF16] + y1f[...]) * dinv + b1f[...], 0.0)
  ho[...] = h * dinv


def _tcc_body(dg, p2f, hdf, w2, b2, o):
  dinv = lax.rsqrt(dg[0, :_NF16] + dg[1, :_NF16] + 1.0)
  f = (p2f[0, :_NF16] + p2f[1, :_NF16] + hdf[...]) * dinv
  for c in range(8):
    fc = f[:, c * 16:(c + 1) * 16]
    a = jnp.dot(fc, w2[...], preferred_element_type=jnp.float32) + b2[...]
    m = jnp.max(a, axis=1, keepdims=True)
    ex = jnp.exp(a - m)
    o[:, c, :] = a - (jnp.log(jnp.sum(ex, axis=1, keepdims=True)) + m)


_tca = pl.pallas_call(
    _tca_body,
    out_shape=jax.ShapeDtypeStruct((_NF16, 128), jnp.float32),
)

_tcb = pl.pallas_call(
    _tcb_body,
    out_shape=jax.ShapeDtypeStruct((_NF16, 128), jnp.float32),
)

_tcc = pl.pallas_call(
    _tcc_body,
    out_shape=jax.ShapeDtypeStruct((_NF16, 8, D_OUT), jnp.float32),
)


def kernel(x, edge_index, W1, b1, W2, b2):
  ei = edge_index.astype(jnp.int32)
  src3 = ei[0].reshape(NW, NB, B)
  dst3 = ei[1].reshape(NW, NB, B)

  _deg, _edge16 = _sc_kernels()
  degp = _deg(dst3)
  degf = degp.reshape(NC, 1280, 128)

  x8 = x.reshape(1250, 1024)
  w1b = jnp.kron(jnp.eye(8, dtype=jnp.float32), W1)
  y1f = _tca(x8, w1b, degf)

  p1 = _edge16(y1f.reshape(N, D_HID), src3, dst3)
  p1f = p1.reshape(NC, 1280, 128)
  b1f = jnp.tile(b1, 8).reshape(1, 128)
  hdf = _tcb(degf, p1f, y1f, b1f)

  p2 = _edge16(hdf.reshape(N, D_HID), src3, dst3)
  p2f = p2.reshape(NC, 1280, 128)
  out3 = _tcc(degf, p2f, hdf, W2, b2.reshape(1, D_OUT))
  return out3.reshape(N, D_OUT)
